# baseline (speedup 1.0000x reference)
import functools

import jax
import jax.numpy as jnp
from jax import lax
from jax.experimental import pallas as pl
from jax.experimental.pallas import tpu as pltpu
from jax.experimental.pallas import tpu_sc as plsc

N = 10000
NFEAT = 128
NHID = 32
NCLASS = 16
NEDGE = 16
T = 4096
CPAD = 128

BM = 400

_INTERPRET = False


def _gcn_kernel(adj_ref, x_ref, w1_ref, b1_ref, w2_ref, b2_ref, o_ref,
                p_s, q_s):
    phase = pl.program_id(0)
    i = pl.program_id(1)

    @pl.when((phase == 0) & (i == 0))
    def _():
        p_s[...] = jnp.dot(x_ref[...], w1_ref[...],
                           preferred_element_type=jnp.float32)

    @pl.when(phase == 0)
    def _():
        acc = jnp.dot(adj_ref[...], p_s[...],
                      preferred_element_type=jnp.float32)
        h = jnp.maximum(acc + b1_ref[...], 0.0)
        q_s[pl.ds(i * BM, BM), :] = jnp.dot(
            h, w2_ref[...], preferred_element_type=jnp.float32)

    @pl.when(phase == 1)
    def _():
        o_ref[...] = jnp.dot(adj_ref[...], q_s[...],
                             preferred_element_type=jnp.float32) + b2_ref[...]


def _gcn(adj2d, x2d, W1, b1r, W2pad, b2pad):
    grid = (2, N // BM)
    return pl.pallas_call(
        _gcn_kernel,
        grid=grid,
        in_specs=[
            pl.BlockSpec((BM, N), lambda p, i: (i, 0)),
            pl.BlockSpec((N, NFEAT), lambda p, i: (0, 0)),
            pl.BlockSpec((NFEAT, NHID), lambda p, i: (0, 0)),
            pl.BlockSpec((1, NHID), lambda p, i: (0, 0)),
            pl.BlockSpec((NHID, CPAD), lambda p, i: (0, 0)),
            pl.BlockSpec((1, CPAD), lambda p, i: (0, 0)),
        ],
        out_specs=pl.BlockSpec((BM, CPAD), lambda p, i: (i * p, 0)),
        out_shape=jax.ShapeDtypeStruct((N, CPAD), jnp.float32),
        scratch_shapes=[
            pltpu.VMEM((N, NHID), jnp.float32),
            pltpu.VMEM((N, CPAD), jnp.float32),
        ],
        interpret=_INTERPRET,
    )(adj2d, x2d, W1, b1r, W2pad, b2pad)


_SC_NC = 2
_SC_NS = 16
_SC_NW = _SC_NC * _SC_NS
_B_PER_W = (3 * T) // _SC_NW


def _gather_rows(table, idx):
    mesh = plsc.VectorSubcoreMesh(core_axis_name="c", subcore_axis_name="s")

    @functools.partial(
        pl.kernel,
        mesh=mesh,
        out_type=jax.ShapeDtypeStruct((3 * T, CPAD), jnp.float32),
        scratch_types=[
            pltpu.VMEM((_B_PER_W,), jnp.int32),
            pltpu.VMEM((_B_PER_W, CPAD), jnp.float32),
            pltpu.SemaphoreType.DMA,
        ],
    )
    def gather_k(idx_hbm, table_hbm, out_hbm, idx_v, rows_v, sem):
        wid = lax.axis_index("s") * _SC_NC + lax.axis_index("c")
        base = wid * _B_PER_W
        pltpu.sync_copy(idx_hbm.at[pl.ds(base, _B_PER_W)], idx_v)
        pltpu.async_copy(table_hbm.at[idx_v], rows_v, sem).wait()
        pltpu.sync_copy(rows_v, out_hbm.at[pl.ds(base, _B_PER_W)])

    return gather_k(idx, table)


def _head_kernel(g_ref, tx_ref, wg_ref, wt_ref, bl_ref, o_ref):
    z = (jnp.dot(g_ref[...], wg_ref[...], preferred_element_type=jnp.float32)
         + jnp.dot(tx_ref[...], wt_ref[...], preferred_element_type=jnp.float32)
         + bl_ref[...])
    z = jnp.maximum(z, 0.0)
    m = jnp.max(z, axis=1, keepdims=True)
    e = jnp.exp(z - m)
    lse = jnp.log(jnp.sum(e, axis=1, keepdims=True)) + m
    o_ref[...] = z - lse


def _head(g48, tx2d, WgT, WtT, blr):
    return pl.pallas_call(
        _head_kernel,
        out_shape=jax.ShapeDtypeStruct((T, NCLASS), jnp.float32),
        interpret=_INTERPRET,
    )(g48, tx2d, WgT, WtT, blr)


def kernel(x, adj, tx, tid, W1, b1, W2, b2, Wl, bl):
    x2d = x.reshape(N, NFEAT)
    adj2d = adj.reshape(N, N)
    tx2d = tx.reshape(T, NEDGE)
    idx = tid.reshape(3 * T)

    W2pad = jnp.pad(W2, ((0, 0), (0, CPAD - NCLASS)))
    b2pad = jnp.pad(b2, (0, CPAD - NCLASS)).reshape(1, CPAD)

    h2 = _gcn(adj2d, x2d, W1, b1.reshape(1, NHID), W2pad, b2pad)

    mark = jnp.zeros((N,), jnp.int32).at[idx].set(1)
    csum = jnp.cumsum(mark)
    posk = csum[idx] - 1
    tgt = jnp.where(mark == 1, csum - 1, N + 7)
    uidx = jnp.zeros((N,), jnp.int32).at[tgt].set(
        jnp.arange(N, dtype=jnp.int32), mode="drop")
    idx_used = jnp.where(posk >= 0, idx, uidx[:12288 // 4].repeat(4) * 0)

    g = _gather_rows(h2, idx_used)
    gwide = g.reshape(T, 3 * CPAD)

    WgT = Wl[:, : 3 * NCLASS].T.reshape(3, NCLASS, NCLASS)
    WgT = jnp.pad(WgT, ((0, 0), (0, CPAD - NCLASS), (0, 0)))
    WgT = WgT.reshape(3 * CPAD, NCLASS)
    WtT = Wl[:, 3 * NCLASS:].T
    return _head(gwide, tx2d, WgT, WtT, bl.reshape(1, NCLASS))


# speedup vs baseline: 1.1140x; 1.1140x over previous
import functools

import jax
import jax.numpy as jnp
from jax import lax
from jax.experimental import pallas as pl
from jax.experimental.pallas import tpu as pltpu
from jax.experimental.pallas import tpu_sc as plsc

N = 10000
NFEAT = 128
NHID = 32
NCLASS = 16
NEDGE = 16
T = 4096
CPAD = 128

BM = 400

_INTERPRET = False


def _gcn_kernel(adj_ref, x_ref, w1_ref, b1_ref, w2_ref, b2_ref, o_ref,
                p_s, q_s):
    phase = pl.program_id(0)
    i = pl.program_id(1)

    @pl.when((phase == 0) & (i == 0))
    def _():
        p_s[...] = jnp.dot(x_ref[...], w1_ref[...],
                           preferred_element_type=jnp.float32)

    @pl.when(phase == 0)
    def _():
        acc = jnp.dot(adj_ref[...], p_s[...],
                      preferred_element_type=jnp.float32)
        h = jnp.maximum(acc + b1_ref[...], 0.0)
        q_s[pl.ds(i * BM, BM), :] = jnp.dot(
            h, w2_ref[...], preferred_element_type=jnp.float32)

    @pl.when(phase == 1)
    def _():
        o_ref[...] = jnp.dot(adj_ref[...], q_s[...],
                             preferred_element_type=jnp.float32) + b2_ref[...]


def _gcn(adj2d, x2d, W1, b1r, W2pad, b2pad):
    grid = (2, N // BM)
    return pl.pallas_call(
        _gcn_kernel,
        grid=grid,
        in_specs=[
            pl.BlockSpec((BM, N), lambda p, i: (i, 0)),
            pl.BlockSpec((N, NFEAT), lambda p, i: (0, 0)),
            pl.BlockSpec((NFEAT, NHID), lambda p, i: (0, 0)),
            pl.BlockSpec((1, NHID), lambda p, i: (0, 0)),
            pl.BlockSpec((NHID, CPAD), lambda p, i: (0, 0)),
            pl.BlockSpec((1, CPAD), lambda p, i: (0, 0)),
        ],
        out_specs=pl.BlockSpec((BM, CPAD), lambda p, i: (i * p, 0)),
        out_shape=jax.ShapeDtypeStruct((N, CPAD), jnp.float32),
        scratch_shapes=[
            pltpu.VMEM((N, NHID), jnp.float32),
            pltpu.VMEM((N, CPAD), jnp.float32),
        ],
        interpret=_INTERPRET,
    )(adj2d, x2d, W1, b1r, W2pad, b2pad)


_SC_NC = 2
_SC_NS = 16
_SC_NW = _SC_NC * _SC_NS
_B_PER_W = (3 * T) // _SC_NW


def _gather_rows(table, idx):
    mesh = plsc.VectorSubcoreMesh(core_axis_name="c", subcore_axis_name="s")

    @functools.partial(
        pl.kernel,
        mesh=mesh,
        out_type=jax.ShapeDtypeStruct((3 * T, CPAD), jnp.float32),
        scratch_types=[
            pltpu.VMEM((_B_PER_W,), jnp.int32),
            pltpu.VMEM((_B_PER_W, CPAD), jnp.float32),
            pltpu.SemaphoreType.DMA,
        ],
    )
    def gather_k(idx_hbm, table_hbm, out_hbm, idx_v, rows_v, sem):
        wid = lax.axis_index("s") * _SC_NC + lax.axis_index("c")
        base = wid * _B_PER_W
        pltpu.sync_copy(idx_hbm.at[pl.ds(base, _B_PER_W)], idx_v)
        pltpu.async_copy(table_hbm.at[idx_v], rows_v, sem).wait()
        pltpu.sync_copy(rows_v, out_hbm.at[pl.ds(base, _B_PER_W)])

    return gather_k(idx, table)


def _head_kernel(g_ref, tx_ref, wg_ref, wt_ref, bl_ref, o_ref):
    z = (jnp.dot(g_ref[...], wg_ref[...], preferred_element_type=jnp.float32)
         + jnp.dot(tx_ref[...], wt_ref[...], preferred_element_type=jnp.float32)
         + bl_ref[...])
    z = jnp.maximum(z, 0.0)
    m = jnp.max(z, axis=1, keepdims=True)
    e = jnp.exp(z - m)
    lse = jnp.log(jnp.sum(e, axis=1, keepdims=True)) + m
    o_ref[...] = z - lse


def _head(g48, tx2d, WgT, WtT, blr):
    return pl.pallas_call(
        _head_kernel,
        out_shape=jax.ShapeDtypeStruct((T, NCLASS), jnp.float32),
        interpret=_INTERPRET,
    )(g48, tx2d, WgT, WtT, blr)


def kernel(x, adj, tx, tid, W1, b1, W2, b2, Wl, bl):
    x2d = x.reshape(N, NFEAT)
    adj2d = adj.reshape(N, N)
    tx2d = tx.reshape(T, NEDGE)
    idx = tid.reshape(3 * T)

    W2pad = jnp.pad(W2, ((0, 0), (0, CPAD - NCLASS)))
    b2pad = jnp.pad(b2, (0, CPAD - NCLASS)).reshape(1, CPAD)

    h2 = _gcn(adj2d, x2d, W1, b1.reshape(1, NHID), W2pad, b2pad)

    gx = _gather_rows(x2d, idx)

    g = _gather_rows(h2, idx) + gx * 0.0
    gwide = g.reshape(T, 3 * CPAD)

    WgT = Wl[:, : 3 * NCLASS].T.reshape(3, NCLASS, NCLASS)
    WgT = jnp.pad(WgT, ((0, 0), (0, CPAD - NCLASS), (0, 0)))
    WgT = WgT.reshape(3 * CPAD, NCLASS)
    WtT = Wl[:, 3 * NCLASS:].T
    return _head(gwide, tx2d, WgT, WtT, bl.reshape(1, NCLASS))


# speedup vs baseline: 1.1827x; 1.0617x over previous
import functools

import jax
import jax.numpy as jnp
from jax import lax
from jax.experimental import pallas as pl
from jax.experimental.pallas import tpu as pltpu
from jax.experimental.pallas import tpu_sc as plsc

N = 10000
NFEAT = 128
NHID = 32
NCLASS = 16
NEDGE = 16
T = 4096
CPAD = 128

BM = 400

_INTERPRET = False


def _gcn_kernel(adj_ref, x_ref, w1_ref, b1_ref, w2_ref, b2_ref, o_ref,
                p_s, q_s):
    phase = pl.program_id(0)
    i = pl.program_id(1)

    @pl.when((phase == 0) & (i == 0))
    def _():
        p_s[...] = jnp.dot(x_ref[...], w1_ref[...],
                           preferred_element_type=jnp.float32)

    @pl.when(phase == 0)
    def _():
        acc = jnp.dot(adj_ref[...], p_s[...],
                      preferred_element_type=jnp.float32)
        h = jnp.maximum(acc + b1_ref[...], 0.0)
        q16 = jnp.dot(h, w2_ref[...], preferred_element_type=jnp.float32)
        q_s[pl.ds(i * BM, BM), :] = jnp.pad(
            q16, ((0, 0), (0, CPAD - NCLASS)))

    @pl.when(phase == 1)
    def _():
        acc = jnp.dot(adj_ref[...], q_s[...],
                      preferred_element_type=jnp.float32)
        b2p = jnp.pad(b2_ref[...], ((0, 0), (0, CPAD - NCLASS)))
        o_ref[...] = acc + b2p


def _gcn(adj2d, x2d, W1, b1r, W2r, b2r):
    grid = (2, N // BM)
    return pl.pallas_call(
        _gcn_kernel,
        grid=grid,
        in_specs=[
            pl.BlockSpec((BM, N), lambda p, i: (i, 0)),
            pl.BlockSpec((N, NFEAT), lambda p, i: (0, 0)),
            pl.BlockSpec((NFEAT, NHID), lambda p, i: (0, 0)),
            pl.BlockSpec((1, NHID), lambda p, i: (0, 0)),
            pl.BlockSpec((NHID, NCLASS), lambda p, i: (0, 0)),
            pl.BlockSpec((1, NCLASS), lambda p, i: (0, 0)),
        ],
        out_specs=pl.BlockSpec((BM, CPAD), lambda p, i: (i * p, 0)),
        out_shape=jax.ShapeDtypeStruct((N, CPAD), jnp.float32),
        scratch_shapes=[
            pltpu.VMEM((N, NHID), jnp.float32),
            pltpu.VMEM((N, CPAD), jnp.float32),
        ],
        interpret=_INTERPRET,
    )(adj2d, x2d, W1, b1r, W2r, b2r)


_SC_NC = 2
_SC_NS = 16
_SC_NW = _SC_NC * _SC_NS
_B_PER_W = (3 * T) // _SC_NW


def _gather_rows(table, idx):
    mesh = plsc.VectorSubcoreMesh(core_axis_name="c", subcore_axis_name="s")

    @functools.partial(
        pl.kernel,
        mesh=mesh,
        out_type=jax.ShapeDtypeStruct((3 * T, CPAD), jnp.float32),
        scratch_types=[
            pltpu.VMEM((_B_PER_W,), jnp.int32),
            pltpu.VMEM((_B_PER_W, CPAD), jnp.float32),
            pltpu.SemaphoreType.DMA,
        ],
    )
    def gather_k(idx_hbm, table_hbm, out_hbm, idx_v, rows_v, sem):
        wid = lax.axis_index("s") * _SC_NC + lax.axis_index("c")
        base = wid * _B_PER_W
        pltpu.sync_copy(idx_hbm.at[pl.ds(base, _B_PER_W)], idx_v)
        pltpu.async_copy(table_hbm.at[idx_v], rows_v, sem).wait()
        pltpu.sync_copy(rows_v, out_hbm.at[pl.ds(base, _B_PER_W)])

    return gather_k(idx, table)


def _head_kernel(g_ref, tx_ref, wl_ref, bl_ref, o_ref):
    dn_t = (((1,), (1,)), ((), ()))
    z = lax.dot_general(tx_ref[...], wl_ref[:, 3 * NCLASS:], dn_t,
                        preferred_element_type=jnp.float32)
    for s in range(3):
        g_s = g_ref[:, s * CPAD: s * CPAD + NCLASS]
        z += lax.dot_general(g_s, wl_ref[:, s * NCLASS:(s + 1) * NCLASS],
                             dn_t, preferred_element_type=jnp.float32)
    z = jnp.maximum(z + bl_ref[...], 0.0)
    m = jnp.max(z, axis=1, keepdims=True)
    e = jnp.exp(z - m)
    lse = jnp.log(jnp.sum(e, axis=1, keepdims=True)) + m
    o_ref[...] = z - lse


def _head(gwide, tx2d, Wl, blr):
    return pl.pallas_call(
        _head_kernel,
        out_shape=jax.ShapeDtypeStruct((T, NCLASS), jnp.float32),
        interpret=_INTERPRET,
    )(gwide, tx2d, Wl, blr)


def kernel(x, adj, tx, tid, W1, b1, W2, b2, Wl, bl):
    x2d = x.reshape(N, NFEAT)
    adj2d = adj.reshape(N, N)
    tx2d = tx.reshape(T, NEDGE)
    idx = tid.reshape(3 * T)

    h2 = _gcn(adj2d, x2d, W1, b1.reshape(1, NHID), W2, b2.reshape(1, NCLASS))

    g = _gather_rows(h2, idx)
    gwide = g.reshape(T, 3 * CPAD)

    return _head(gwide, tx2d, Wl, bl.reshape(1, NCLASS))


# speedup vs baseline: 1.1832x; 1.0005x over previous
import functools

import jax
import jax.numpy as jnp
from jax import lax
from jax.experimental import pallas as pl
from jax.experimental.pallas import tpu as pltpu
from jax.experimental.pallas import tpu_sc as plsc

N = 10000
NFEAT = 128
NHID = 32
NCLASS = 16
NEDGE = 16
T = 4096
CPAD = 128

BM = 400


def _gcn_kernel(adj_ref, x_ref, w1_ref, b1_ref, w2_ref, b2_ref, o_ref,
                p_s, q_s):
    phase = pl.program_id(0)
    i = pl.program_id(1)

    @pl.when((phase == 0) & (i == 0))
    def _():
        p_s[...] = jnp.dot(x_ref[...], w1_ref[...],
                           preferred_element_type=jnp.float32)

    @pl.when(phase == 0)
    def _():
        acc = jnp.dot(adj_ref[...], p_s[...],
                      preferred_element_type=jnp.float32)
        h = jnp.maximum(acc + b1_ref[...], 0.0)
        q16 = jnp.dot(h, w2_ref[...], preferred_element_type=jnp.float32)
        q_s[pl.ds(i * BM, BM), :] = jnp.pad(
            q16, ((0, 0), (0, CPAD - NCLASS)))

    @pl.when(phase == 1)
    def _():
        acc = jnp.dot(adj_ref[...], q_s[...],
                      preferred_element_type=jnp.float32)
        b2p = jnp.pad(b2_ref[...], ((0, 0), (0, CPAD - NCLASS)))
        o_ref[...] = acc + b2p


def _gcn(adj2d, x2d, W1, b1r, W2r, b2r):
    grid = (2, N // BM)
    return pl.pallas_call(
        _gcn_kernel,
        grid=grid,
        in_specs=[
            pl.BlockSpec((BM, N), lambda p, i: (i, 0)),
            pl.BlockSpec((N, NFEAT), lambda p, i: (0, 0)),
            pl.BlockSpec((NFEAT, NHID), lambda p, i: (0, 0)),
            pl.BlockSpec((1, NHID), lambda p, i: (0, 0)),
            pl.BlockSpec((NHID, NCLASS), lambda p, i: (0, 0)),
            pl.BlockSpec((1, NCLASS), lambda p, i: (0, 0)),
        ],
        out_specs=pl.BlockSpec((BM, CPAD), lambda p, i: (i * p, 0)),
        out_shape=jax.ShapeDtypeStruct((N, CPAD), jnp.float32),
        scratch_shapes=[
            pltpu.VMEM((N, NHID), jnp.float32),
            pltpu.VMEM((N, CPAD), jnp.float32),
        ],
    )(adj2d, x2d, W1, b1r, W2r, b2r)


_SC_NC = 2
_SC_NS = 16
_SC_NW = _SC_NC * _SC_NS
_B_PER_W = (3 * T) // _SC_NW


def _gather_rows(table, idx):
    mesh = plsc.VectorSubcoreMesh(core_axis_name="c", subcore_axis_name="s")

    @functools.partial(
        pl.kernel,
        mesh=mesh,
        out_type=jax.ShapeDtypeStruct((3 * T, CPAD), jnp.float32),
        scratch_types=[
            pltpu.VMEM((_B_PER_W,), jnp.int32),
            pltpu.VMEM((_B_PER_W, CPAD), jnp.float32),
            pltpu.SemaphoreType.DMA,
        ],
    )
    def gather_k(idx_hbm, table_hbm, out_hbm, idx_v, rows_v, sem):
        wid = lax.axis_index("s") * _SC_NC + lax.axis_index("c")
        base = wid * _B_PER_W
        pltpu.sync_copy(idx_hbm.at[pl.ds(base, _B_PER_W)], idx_v)
        pltpu.async_copy(table_hbm.at[idx_v], rows_v, sem).wait()
        pltpu.sync_copy(rows_v, out_hbm.at[pl.ds(base, _B_PER_W)])

    return gather_k(idx, table)


def _head_kernel(g_ref, tx_ref, wl_ref, bl_ref, o_ref):
    dn_t = (((1,), (1,)), ((), ()))
    z = lax.dot_general(tx_ref[...], wl_ref[:, 3 * NCLASS:], dn_t,
                        preferred_element_type=jnp.float32)
    for s in range(3):
        g_s = g_ref[:, s * CPAD: s * CPAD + NCLASS]
        z += lax.dot_general(g_s, wl_ref[:, s * NCLASS:(s + 1) * NCLASS],
                             dn_t, preferred_element_type=jnp.float32)
    z = jnp.maximum(z + bl_ref[...], 0.0)
    m = jnp.max(z, axis=1, keepdims=True)
    e = jnp.exp(z - m)
    lse = jnp.log(jnp.sum(e, axis=1, keepdims=True)) + m
    o_ref[...] = z - lse


def _head(gwide, tx2d, Wl, blr):
    return pl.pallas_call(
        _head_kernel,
        out_shape=jax.ShapeDtypeStruct((T, NCLASS), jnp.float32),
    )(gwide, tx2d, Wl, blr)


def kernel(x, adj, tx, tid, W1, b1, W2, b2, Wl, bl):
    x2d = x.reshape(N, NFEAT)
    adj2d = adj.reshape(N, N)
    tx2d = tx.reshape(T, NEDGE)
    idx = tid.reshape(3 * T)

    h2 = _gcn(adj2d, x2d, W1, b1.reshape(1, NHID), W2, b2.reshape(1, NCLASS))

    g = _gather_rows(h2, idx)
    gwide = g.reshape(T, 3 * CPAD)

    return _head(gwide, tx2d, Wl, bl.reshape(1, NCLASS))
